# baseline (speedup 1.0000x reference)
import functools

import jax
import jax.numpy as jnp
from jax.experimental import pallas as pl

N = 10000
D = 128
BM = 400


def _gin_block_kernel(adj_ref, hfull_ref, hblk_ref, scale_ref,
                      w1_ref, c1_ref, w2_ref, c2_ref, out_ref):
    a_bf = adj_ref[...].astype(jnp.bfloat16)
    pooled = jax.lax.dot_general(
        a_bf, hfull_ref[...], (((1,), (0,)), ((), ())),
        preferred_element_type=jnp.float32)
    pooled = pooled + scale_ref[0, 0] * hblk_ref[...].astype(jnp.float32)
    t = jax.lax.dot_general(
        pooled, w1_ref[...], (((1,), (0,)), ((), ())),
        preferred_element_type=jnp.float32,
        precision=jax.lax.Precision.HIGHEST)
    t = jnp.maximum(t + c1_ref[...], 0.0)
    t = jax.lax.dot_general(
        t, w2_ref[...], (((1,), (0,)), ((), ())),
        preferred_element_type=jnp.float32,
        precision=jax.lax.Precision.HIGHEST)
    h_out = jnp.maximum(t + c2_ref[...], 0.0)
    out_ref[...] = h_out.astype(out_ref.dtype)


def _gin_head_kernel(adj_ref, hfull_ref, hblk_ref, scale_ref,
                     w1_ref, c1_ref, w2_ref, c2_ref, wpt_ref, bp_ref,
                     out_ref):
    a_bf = adj_ref[...].astype(jnp.bfloat16)
    pooled = jax.lax.dot_general(
        a_bf, hfull_ref[...], (((1,), (0,)), ((), ())),
        preferred_element_type=jnp.float32)
    pooled = pooled + scale_ref[0, 0] * hblk_ref[...].astype(jnp.float32)
    t = jax.lax.dot_general(
        pooled, w1_ref[...], (((1,), (0,)), ((), ())),
        preferred_element_type=jnp.float32,
        precision=jax.lax.Precision.HIGHEST)
    t = jnp.maximum(t + c1_ref[...], 0.0)
    t = jax.lax.dot_general(
        t, w2_ref[...], (((1,), (0,)), ((), ())),
        preferred_element_type=jnp.float32,
        precision=jax.lax.Precision.HIGHEST)
    h2 = jnp.maximum(t + c2_ref[...], 0.0)
    score = jnp.sum(h2 * wpt_ref[...], axis=1, keepdims=True) + bp_ref[0, 0]
    out_ref[...] = score


def _layer_specs(n, bm, d, extra_in=0):
    grid = (n // bm,)
    in_specs = [
        pl.BlockSpec((bm, n), lambda i: (i, 0)),
        pl.BlockSpec((n, d), lambda i: (0, 0)),
        pl.BlockSpec((bm, d), lambda i: (i, 0)),
        pl.BlockSpec((1, 1), lambda i: (0, 0)),
        pl.BlockSpec((d, d), lambda i: (0, 0)),
        pl.BlockSpec((1, d), lambda i: (0, 0)),
        pl.BlockSpec((d, d), lambda i: (0, 0)),
        pl.BlockSpec((1, d), lambda i: (0, 0)),
    ]
    if extra_in:
        in_specs += [
            pl.BlockSpec((1, d), lambda i: (0, 0)),
            pl.BlockSpec((1, 1), lambda i: (0, 0)),
        ]
    return grid, in_specs


def _fold_bn(W, b, g, beta):
    s = g * (1.0 + 1e-5) ** -0.5
    return W * s[None, :], (b * s + beta)[None, :]


@functools.partial(jax.jit, static_argnames=())
def _run(seq1, adj, eps,
         l0_W1, l0_b1, l0_bn1_g, l0_bn1_b, l0_W2, l0_b2, l0_bn2_g, l0_bn2_b,
         l1_W1, l1_b1, l1_bn1_g, l1_bn1_b, l1_W2, l1_b2, l1_bn2_g, l1_bn2_b,
         Wp, bp):
    n, d = seq1.shape
    bm = BM if n % BM == 0 else n
    w10, c10 = _fold_bn(l0_W1, l0_b1, l0_bn1_g, l0_bn1_b)
    w20, c20 = _fold_bn(l0_W2, l0_b2, l0_bn2_g, l0_bn2_b)
    w11, c11 = _fold_bn(l1_W1, l1_b1, l1_bn1_g, l1_bn1_b)
    w21, c21 = _fold_bn(l1_W2, l1_b2, l1_bn2_g, l1_bn2_b)
    s0 = (1.0 + eps[0]).reshape(1, 1)
    s1 = (1.0 + eps[1]).reshape(1, 1)
    h0 = seq1.astype(jnp.bfloat16)

    grid, in_specs = _layer_specs(n, bm, d)
    h1 = pl.pallas_call(
        _gin_block_kernel,
        grid=grid,
        in_specs=in_specs,
        out_specs=pl.BlockSpec((bm, d), lambda i: (i, 0)),
        out_shape=jax.ShapeDtypeStruct((n, d), jnp.bfloat16),
    )(adj, h0, h0, s0, w10, c10, w20, c20)

    grid, in_specs = _layer_specs(n, bm, d, extra_in=1)
    score = pl.pallas_call(
        _gin_head_kernel,
        grid=grid,
        in_specs=in_specs,
        out_specs=pl.BlockSpec((bm, 1), lambda i: (i, 0)),
        out_shape=jax.ShapeDtypeStruct((n, 1), jnp.float32),
    )(adj, h1, h1, s1, w11, c11, w21, c21, Wp.T, bp.reshape(1, 1))
    return score


def kernel(seq1, adj, eps,
           l0_W1, l0_b1, l0_bn1_g, l0_bn1_b, l0_W2, l0_b2, l0_bn2_g, l0_bn2_b,
           l1_W1, l1_b1, l1_bn1_g, l1_bn1_b, l1_W2, l1_b2, l1_bn2_g, l1_bn2_b,
           Wp, bp):
    return _run(seq1, adj, eps,
                l0_W1, l0_b1, l0_bn1_g, l0_bn1_b, l0_W2, l0_b2, l0_bn2_g,
                l0_bn2_b, l1_W1, l1_b1, l1_bn1_g, l1_bn1_b, l1_W2, l1_b2,
                l1_bn2_g, l1_bn2_b, Wp, bp)


# speedup vs baseline: 1.0145x; 1.0145x over previous
import functools

import jax
import jax.numpy as jnp
from jax.experimental import pallas as pl

N = 10000
D = 128
BM = 400


def _gin_block_kernel(adj_ref, hfull_ref, hblk_ref, scale_ref,
                      w1_ref, c1_ref, w2_ref, c2_ref, out_ref):
    pooled = jax.lax.dot_general(
        adj_ref[...], hfull_ref[...], (((1,), (0,)), ((), ())),
        preferred_element_type=jnp.float32)
    pooled = pooled + scale_ref[0, 0] * hblk_ref[...].astype(jnp.float32)
    t = jax.lax.dot_general(
        pooled, w1_ref[...], (((1,), (0,)), ((), ())),
        preferred_element_type=jnp.float32)
    t = jnp.maximum(t + c1_ref[...], 0.0)
    t = jax.lax.dot_general(
        t, w2_ref[...], (((1,), (0,)), ((), ())),
        preferred_element_type=jnp.float32)
    h_out = jnp.maximum(t + c2_ref[...], 0.0)
    out_ref[...] = h_out.astype(out_ref.dtype)


def _gin_head_kernel(adj_ref, hfull_ref, hblk_ref, scale_ref,
                     w1_ref, c1_ref, w2_ref, c2_ref, wpt_ref, bp_ref,
                     out_ref):
    pooled = jax.lax.dot_general(
        adj_ref[...], hfull_ref[...], (((1,), (0,)), ((), ())),
        preferred_element_type=jnp.float32)
    pooled = pooled + scale_ref[0, 0] * hblk_ref[...].astype(jnp.float32)
    t = jax.lax.dot_general(
        pooled, w1_ref[...], (((1,), (0,)), ((), ())),
        preferred_element_type=jnp.float32)
    t = jnp.maximum(t + c1_ref[...], 0.0)
    t = jax.lax.dot_general(
        t, w2_ref[...], (((1,), (0,)), ((), ())),
        preferred_element_type=jnp.float32)
    h2 = jnp.maximum(t + c2_ref[...], 0.0)
    score = jnp.sum(h2 * wpt_ref[...], axis=1, keepdims=True) + bp_ref[0, 0]
    out_ref[...] = score


def _layer_specs(n, bm, d, extra_in=0):
    grid = (n // bm,)
    in_specs = [
        pl.BlockSpec((bm, n), lambda i: (i, 0)),
        pl.BlockSpec((n, d), lambda i: (0, 0)),
        pl.BlockSpec((bm, d), lambda i: (i, 0)),
        pl.BlockSpec((1, 1), lambda i: (0, 0)),
        pl.BlockSpec((d, d), lambda i: (0, 0)),
        pl.BlockSpec((1, d), lambda i: (0, 0)),
        pl.BlockSpec((d, d), lambda i: (0, 0)),
        pl.BlockSpec((1, d), lambda i: (0, 0)),
    ]
    if extra_in:
        in_specs += [
            pl.BlockSpec((1, d), lambda i: (0, 0)),
            pl.BlockSpec((1, 1), lambda i: (0, 0)),
        ]
    return grid, in_specs


def _fold_bn(W, b, g, beta):
    s = g * (1.0 + 1e-5) ** -0.5
    return W * s[None, :], (b * s + beta)[None, :]


@functools.partial(jax.jit, static_argnames=())
def _run(seq1, adj, eps,
         l0_W1, l0_b1, l0_bn1_g, l0_bn1_b, l0_W2, l0_b2, l0_bn2_g, l0_bn2_b,
         l1_W1, l1_b1, l1_bn1_g, l1_bn1_b, l1_W2, l1_b2, l1_bn2_g, l1_bn2_b,
         Wp, bp):
    n, d = seq1.shape
    bm = BM if n % BM == 0 else n
    w10, c10 = _fold_bn(l0_W1, l0_b1, l0_bn1_g, l0_bn1_b)
    w20, c20 = _fold_bn(l0_W2, l0_b2, l0_bn2_g, l0_bn2_b)
    w11, c11 = _fold_bn(l1_W1, l1_b1, l1_bn1_g, l1_bn1_b)
    w21, c21 = _fold_bn(l1_W2, l1_b2, l1_bn2_g, l1_bn2_b)
    s0 = (1.0 + eps[0]).reshape(1, 1)
    s1 = (1.0 + eps[1]).reshape(1, 1)
    h0 = seq1

    grid, in_specs = _layer_specs(n, bm, d)
    h1 = pl.pallas_call(
        _gin_block_kernel,
        grid=grid,
        in_specs=in_specs,
        out_specs=pl.BlockSpec((bm, d), lambda i: (i, 0)),
        out_shape=jax.ShapeDtypeStruct((n, d), jnp.float32),
    )(adj, h0, h0, s0, w10, c10, w20, c20)

    grid, in_specs = _layer_specs(n, bm, d, extra_in=1)
    score = pl.pallas_call(
        _gin_head_kernel,
        grid=grid,
        in_specs=in_specs,
        out_specs=pl.BlockSpec((bm, 1), lambda i: (i, 0)),
        out_shape=jax.ShapeDtypeStruct((n, 1), jnp.float32),
    )(adj, h1, h1, s1, w11, c11, w21, c21, Wp.T, bp.reshape(1, 1))
    return score


def kernel(seq1, adj, eps,
           l0_W1, l0_b1, l0_bn1_g, l0_bn1_b, l0_W2, l0_b2, l0_bn2_g, l0_bn2_b,
           l1_W1, l1_b1, l1_bn1_g, l1_bn1_b, l1_W2, l1_b2, l1_bn2_g, l1_bn2_b,
           Wp, bp):
    return _run(seq1, adj, eps,
                l0_W1, l0_b1, l0_bn1_g, l0_bn1_b, l0_W2, l0_b2, l0_bn2_g,
                l0_bn2_b, l1_W1, l1_b1, l1_bn1_g, l1_bn1_b, l1_W2, l1_b2,
                l1_bn2_g, l1_bn2_b, Wp, bp)


# speedup vs baseline: 1.0524x; 1.0373x over previous
import functools

import jax
import jax.numpy as jnp
from jax.experimental import pallas as pl

BM = 400
_BN_RSQRT = (1.0 + 1e-5) ** -0.5


def _mlp(pooled, w1_ref, b1_ref, g1_ref, bt1_ref, w2_ref, b2_ref, g2_ref,
         bt2_ref):
    s1 = g1_ref[...] * _BN_RSQRT
    t = jax.lax.dot_general(
        pooled, w1_ref[...], (((1,), (0,)), ((), ())),
        preferred_element_type=jnp.float32)
    t = jnp.maximum(t * s1 + (b1_ref[...] * s1 + bt1_ref[...]), 0.0)
    s2 = g2_ref[...] * _BN_RSQRT
    t = jax.lax.dot_general(
        t, w2_ref[...], (((1,), (0,)), ((), ())),
        preferred_element_type=jnp.float32)
    return jnp.maximum(t * s2 + (b2_ref[...] * s2 + bt2_ref[...]), 0.0)


def _gin_layer_kernel(adj_ref, hfull_ref, hblk_ref, eps_ref,
                      w1_ref, b1_ref, g1_ref, bt1_ref,
                      w2_ref, b2_ref, g2_ref, bt2_ref, out_ref):
    pooled = jax.lax.dot_general(
        adj_ref[...], hfull_ref[...], (((1,), (0,)), ((), ())),
        preferred_element_type=jnp.float32)
    pooled = pooled + (1.0 + eps_ref[0, 0]) * hblk_ref[...]
    out_ref[...] = _mlp(pooled, w1_ref, b1_ref, g1_ref, bt1_ref,
                        w2_ref, b2_ref, g2_ref, bt2_ref)


def _gin_head_kernel(adj_ref, hfull_ref, hblk_ref, eps_ref,
                     w1_ref, b1_ref, g1_ref, bt1_ref,
                     w2_ref, b2_ref, g2_ref, bt2_ref,
                     wp_ref, bp_ref, out_ref):
    pooled = jax.lax.dot_general(
        adj_ref[...], hfull_ref[...], (((1,), (0,)), ((), ())),
        preferred_element_type=jnp.float32)
    pooled = pooled + (1.0 + eps_ref[0, 1]) * hblk_ref[...]
    h2 = _mlp(pooled, w1_ref, b1_ref, g1_ref, bt1_ref,
              w2_ref, b2_ref, g2_ref, bt2_ref)
    score = jax.lax.dot_general(
        h2, wp_ref[...], (((1,), (0,)), ((), ())),
        preferred_element_type=jnp.float32)
    out_ref[...] = score + bp_ref[0, 0]


def _layer_specs(n, bm, d, head):
    grid = (n // bm,)
    full = lambda i: (0, 0)
    vec = pl.BlockSpec((1, d), full)
    mat = pl.BlockSpec((d, d), full)
    in_specs = [
        pl.BlockSpec((bm, n), lambda i: (i, 0)),
        pl.BlockSpec((n, d), full),
        pl.BlockSpec((bm, d), lambda i: (i, 0)),
        pl.BlockSpec((1, 2), full),
        mat, vec, vec, vec,
        mat, vec, vec, vec,
    ]
    if head:
        in_specs += [
            pl.BlockSpec((d, 1), full),
            pl.BlockSpec((1, 1), full),
        ]
    return grid, in_specs


@jax.jit
def _run(seq1, adj, eps,
         l0_W1, l0_b1, l0_bn1_g, l0_bn1_b, l0_W2, l0_b2, l0_bn2_g, l0_bn2_b,
         l1_W1, l1_b1, l1_bn1_g, l1_bn1_b, l1_W2, l1_b2, l1_bn2_g, l1_bn2_b,
         Wp, bp):
    n, d = seq1.shape
    bm = BM if n % BM == 0 else n
    r = lambda v: v.reshape(1, d)
    eps2 = eps.reshape(1, 2)

    grid, in_specs = _layer_specs(n, bm, d, head=False)
    h1 = pl.pallas_call(
        _gin_layer_kernel,
        grid=grid,
        in_specs=in_specs,
        out_specs=pl.BlockSpec((bm, d), lambda i: (i, 0)),
        out_shape=jax.ShapeDtypeStruct((n, d), jnp.float32),
    )(adj, seq1, seq1, eps2,
      l0_W1, r(l0_b1), r(l0_bn1_g), r(l0_bn1_b),
      l0_W2, r(l0_b2), r(l0_bn2_g), r(l0_bn2_b))

    grid, in_specs = _layer_specs(n, bm, d, head=True)
    score = pl.pallas_call(
        _gin_head_kernel,
        grid=grid,
        in_specs=in_specs,
        out_specs=pl.BlockSpec((bm, 1), lambda i: (i, 0)),
        out_shape=jax.ShapeDtypeStruct((n, 1), jnp.float32),
    )(adj, h1, h1, eps2,
      l1_W1, r(l1_b1), r(l1_bn1_g), r(l1_bn1_b),
      l1_W2, r(l1_b2), r(l1_bn2_g), r(l1_bn2_b),
      Wp, bp.reshape(1, 1))
    return score


def kernel(seq1, adj, eps,
           l0_W1, l0_b1, l0_bn1_g, l0_bn1_b, l0_W2, l0_b2, l0_bn2_g, l0_bn2_b,
           l1_W1, l1_b1, l1_bn1_g, l1_bn1_b, l1_W2, l1_b2, l1_bn2_g, l1_bn2_b,
           Wp, bp):
    return _run(seq1, adj, eps,
                l0_W1, l0_b1, l0_bn1_g, l0_bn1_b, l0_W2, l0_b2, l0_bn2_g,
                l0_bn2_b, l1_W1, l1_b1, l1_bn1_g, l1_bn1_b, l1_W2, l1_b2,
                l1_bn2_g, l1_bn2_b, Wp, bp)
